# baseline (speedup 1.0000x reference)
import jax
import jax.numpy as jnp
import numpy as np
from jax.experimental import pallas as pl


def _sqdist(a, b):
    return jnp.sum(a * a, -1)[:, :, None] + jnp.sum(b * b, -1)[:, None, :] - 2.0 * jnp.einsum('bnd,bmd->bnm', a, b)


def _gather_points(x, idx):
    return jax.vmap(lambda xi, ii: xi[ii])(x, idx)


def _fps(xyz, npoint):
    B, N, _ = xyz.shape
    def body(i, carry):
        dist, far, inds = carry
        inds = inds.at[:, i].set(far)
        centroid = jax.vmap(lambda x, f: x[f])(xyz, far)[:, None, :]
        d = jnp.sum((xyz - centroid) ** 2, -1)
        dist = jnp.minimum(dist, d)
        far = jnp.argmax(dist, axis=-1).astype(jnp.int32)
        return dist, far, inds
    carry = (jnp.full((B, N), 1e10, jnp.float32), jnp.zeros((B,), jnp.int32), jnp.zeros((B, npoint), jnp.int32))
    _, _, inds = jax.lax.fori_loop(0, npoint, body, carry)
    return inds


def _ball_query(radius, nsample, xyz, new_xyz):
    N = xyz.shape[1]
    d2 = _sqdist(new_xyz, xyz)
    mask = d2 <= radius * radius
    key = jnp.where(mask, jnp.arange(N, dtype=jnp.int32)[None, None, :], N)
    _, idx = jax.lax.top_k(-key, nsample)
    cnt = jnp.sum(mask, -1, keepdims=True)
    idx = jnp.where(jnp.arange(nsample)[None, None, :] < cnt, idx, idx[..., :1])
    return idx


def _shared_mlp(g, layers):
    for (W, gamma, beta) in layers:
        g = jnp.einsum('oc,bcns->bons', W, g)
        g = g / np.sqrt(1.0 + 1e-5) * gamma[None, :, None, None] + beta[None, :, None, None]
        g = jax.nn.relu(g)
    return g


def _sa_module(xyz, features, npoint, radius, nsample, layers):
    inds = _fps(xyz, npoint)
    new_xyz = _gather_points(xyz, inds)
    idx = _ball_query(radius, nsample, xyz, new_xyz)
    grouped_xyz = (_gather_points(xyz, idx) - new_xyz[:, :, None, :]) / radius
    g = jnp.transpose(grouped_xyz, (0, 3, 1, 2))
    if features is not None:
        gf = _gather_points(jnp.transpose(features, (0, 2, 1)), idx)
        g = jnp.concatenate([g, jnp.transpose(gf, (0, 3, 1, 2))], axis=1)
    g = _shared_mlp(g, layers)
    return new_xyz, jnp.max(g, axis=-1), inds


def _fp_module(xyz1, xyz2, feats1, feats2, layers):
    d2 = _sqdist(xyz1, xyz2)
    negd, idx = jax.lax.top_k(-d2, 3)
    dist = jax.lax.stop_gradient(jnp.maximum(-negd, 0.0))
    dist_recip = 1.0 / (dist + 1e-8)
    weight = dist_recip / jnp.sum(dist_recip, -1, keepdims=True)
    gf = _gather_points(jnp.transpose(feats2, (0, 2, 1)), idx)
    interp = jnp.transpose(jnp.sum(gf * weight[..., None], axis=2), (0, 2, 1))
    new = jnp.concatenate([feats1, interp], axis=1) if feats1 is not None else interp
    return _shared_mlp(new[..., None], layers)[..., 0]


def kernel(pointcloud, params):
    xyz = pointcloud[..., 0:3]
    sa1_xyz, sa1_f, sa1_inds = _sa_module(xyz, None, 512, 0.04, 64, params['sa1'])
    sa2_xyz, sa2_f, _ = _sa_module(sa1_xyz, sa1_f, 256, 0.1, 32, params['sa2'])
    sa3_xyz, sa3_f, _ = _sa_module(sa2_xyz, sa2_f, 128, 0.2, 16, params['sa3'])
    sa4_xyz, sa4_f, _ = _sa_module(sa3_xyz, sa3_f, 64, 0.3, 16, params['sa4'])
    f = _fp_module(sa3_xyz, sa4_xyz, sa3_f, sa4_f, params['fp1'])
    f = _fp_module(sa2_xyz, sa3_xyz, sa2_f, f, params['fp2'])
    f = _fp_module(sa1_xyz, sa2_xyz, sa1_f, f, params['fp3'])
    return f, sa1_xyz, sa1_inds


# speedup vs baseline: 1.4501x; 1.4501x over previous
import functools

import jax
import jax.numpy as jnp
import numpy as np
from jax.experimental import pallas as pl
from jax.experimental.pallas import tpu as pltpu

_BN_C = float(np.sqrt(1.0 + 1e-5))
_BIG_I32 = np.int32(2 ** 30)


def _fps_body(npoint, n_valid, xs_ref, ys_ref, zs_ref, inds_ref, dist_ref):
    B, NPAD = xs_ref.shape
    flat = jax.lax.broadcasted_iota(jnp.int32, (B, NPAD), 1)
    valid = flat < n_valid
    xsv = xs_ref[...]
    ysv = ys_ref[...]
    zsv = zs_ref[...]
    dist_ref[...] = jnp.where(valid, jnp.float32(1e10), jnp.float32(-1.0))

    lane_np = jax.lax.broadcasted_iota(jnp.int32, (B, npoint), 1)
    inds_ref[...] = jnp.zeros((B, npoint), jnp.int32)

    def step(i, far):
        inds_ref[...] = jnp.where(lane_np == i, far, inds_ref[...])
        eq = flat == far
        cx = jnp.sum(jnp.where(eq, xsv, 0.0), axis=1, keepdims=True)
        cy = jnp.sum(jnp.where(eq, ysv, 0.0), axis=1, keepdims=True)
        cz = jnp.sum(jnp.where(eq, zsv, 0.0), axis=1, keepdims=True)
        dx = xsv - cx
        dy = ysv - cy
        dz = zsv - cz
        d = (dx * dx + dy * dy) + dz * dz
        dist = jnp.minimum(dist_ref[...], d)
        dist_ref[...] = dist
        m = jnp.max(dist, axis=1, keepdims=True)
        far2 = jnp.min(jnp.where(dist == m, flat, _BIG_I32), axis=1, keepdims=True)
        return far2

    far0 = jnp.zeros((B, 1), jnp.int32)
    jax.lax.fori_loop(0, npoint, step, far0)


def _fps(xyz, npoint, interpret=False):
    B, N, _ = xyz.shape
    NPAD = ((N + 127) // 128) * 128
    xyzp = jnp.pad(xyz, ((0, 0), (0, NPAD - N), (0, 0)))
    xs, ys, zs = xyzp[..., 0], xyzp[..., 1], xyzp[..., 2]
    return pl.pallas_call(
        functools.partial(_fps_body, npoint, N),
        out_shape=jax.ShapeDtypeStruct((B, npoint), jnp.int32),
        scratch_shapes=[pltpu.VMEM((B, NPAD), jnp.float32)],
        interpret=interpret,
    )(xs, ys, zs)


def _bq_body(r2, K, q_ref, p_ref, idx_ref):
    n = q_ref.shape[1]
    N = p_ref.shape[2]
    qx, qy, qz = (q_ref[0, :, i:i + 1] for i in range(3))
    px, py, pz = (p_ref[0, i:i + 1, :] for i in range(3))
    q2 = (qx * qx + qy * qy) + qz * qz
    p2 = (px * px + py * py) + pz * pz
    ab = (qx * px + qy * py) + qz * pz
    d2 = (q2 + p2) - 2.0 * ab
    mask = d2 <= r2
    rank = mask.astype(jnp.int32)
    sh = 1
    while sh < N:
        rank = rank + jnp.concatenate(
            [jnp.zeros((n, sh), jnp.int32), rank[:, :N - sh]], axis=1)
        sh *= 2
    flat = jax.lax.broadcasted_iota(jnp.int32, (n, N), 1)
    firstidx = jnp.min(jnp.where(mask, flat, _BIG_I32), axis=1, keepdims=True)
    cols = []
    for s in range(K):
        sel = mask & (rank == (s + 1))
        cand = jnp.min(jnp.where(sel, flat, _BIG_I32), axis=1, keepdims=True)
        cols.append(jnp.where(cand == _BIG_I32, firstidx, cand))
    idx_ref[0] = jnp.concatenate(cols, axis=1)


def _ball_query_small(radius, K, xyz, new_xyz, interpret=False):
    B, N, _ = xyz.shape
    n = new_xyz.shape[1]
    return pl.pallas_call(
        functools.partial(_bq_body, radius * radius, K),
        grid=(B,),
        in_specs=[
            pl.BlockSpec((1, n, 3), lambda b: (b, 0, 0)),
            pl.BlockSpec((1, 3, N), lambda b: (b, 0, 0)),
        ],
        out_specs=pl.BlockSpec((1, n, K), lambda b: (b, 0, 0)),
        out_shape=jax.ShapeDtypeStruct((B, n, K), jnp.int32),
        interpret=interpret,
    )(new_xyz, jnp.transpose(xyz, (0, 2, 1)))


def _ball_query_xla(radius, nsample, xyz, new_xyz):
    N = xyz.shape[1]
    d2 = (jnp.sum(new_xyz * new_xyz, -1)[:, :, None]
          + jnp.sum(xyz * xyz, -1)[:, None, :]
          - 2.0 * jnp.einsum('bnd,bmd->bnm', new_xyz, xyz))
    mask = d2 <= radius * radius
    key = jnp.where(mask, jnp.arange(N, dtype=jnp.int32)[None, None, :], N)
    _, idx = jax.lax.top_k(-key, nsample)
    cnt = jnp.sum(mask, -1, keepdims=True)
    idx = jnp.where(jnp.arange(nsample)[None, None, :] < cnt, idx, idx[..., :1])
    return idx


def _mlp_max_body(n, K, CH, x_ref, w1_ref, w2_ref, w3_ref, sb_ref, out_ref):
    C3 = w3_ref.shape[0]
    R = x_ref.shape[2]
    nchunk = R // CH
    planes = CH // n
    acc = jnp.full((C3, n), -jnp.inf, jnp.float32)
    w1 = w1_ref[...]
    w2 = w2_ref[...]
    w3 = w3_ref[...]
    s1 = sb_ref[0, 0:1, :w1.shape[0]].reshape(w1.shape[0], 1)
    b1 = sb_ref[0, 1:2, :w1.shape[0]].reshape(w1.shape[0], 1)
    s2 = sb_ref[0, 2:3, :w2.shape[0]].reshape(w2.shape[0], 1)
    b2 = sb_ref[0, 3:4, :w2.shape[0]].reshape(w2.shape[0], 1)
    s3 = sb_ref[0, 4:5, :C3].reshape(C3, 1)
    b3 = sb_ref[0, 5:6, :C3].reshape(C3, 1)
    for c in range(nchunk):
        x = x_ref[0, :, c * CH:(c + 1) * CH]
        z = jnp.dot(w1, x, preferred_element_type=jnp.float32,
                    precision=jax.lax.Precision.HIGHEST)
        z = jnp.maximum(z / _BN_C * s1 + b1, 0.0)
        z = jnp.dot(w2, z, preferred_element_type=jnp.float32,
                    precision=jax.lax.Precision.HIGHEST)
        z = jnp.maximum(z / _BN_C * s2 + b2, 0.0)
        z = jnp.dot(w3, z, preferred_element_type=jnp.float32,
                    precision=jax.lax.Precision.HIGHEST)
        z = jnp.maximum(z / _BN_C * s3 + b3, 0.0)
        for j in range(planes):
            acc = jnp.maximum(acc, z[:, j * n:(j + 1) * n])
    out_ref[0] = acc


def _mlp_max(X, layers, n, K, interpret=False):
    B, Cp, R = X.shape
    (W1, g1, be1), (W2, g2, be2), (W3, g3, be3) = layers
    C1, C2, C3 = W1.shape[0], W2.shape[0], W3.shape[0]
    W1p = jnp.pad(W1, ((0, 0), (0, Cp - W1.shape[1])))
    Cmax = max(C1, C2, C3)
    sb = jnp.zeros((6, Cmax), jnp.float32)
    sb = sb.at[0, :C1].set(g1).at[1, :C1].set(be1)
    sb = sb.at[2, :C2].set(g2).at[3, :C2].set(be2)
    sb = sb.at[4, :C3].set(g3).at[5, :C3].set(be3)
    CH = min(R, 2048)
    return pl.pallas_call(
        functools.partial(_mlp_max_body, n, K, CH),
        grid=(B,),
        in_specs=[
            pl.BlockSpec((1, Cp, R), lambda b: (b, 0, 0)),
            pl.BlockSpec(W1p.shape, lambda b: (0, 0)),
            pl.BlockSpec(W2.shape, lambda b: (0, 0)),
            pl.BlockSpec(W3.shape, lambda b: (0, 0)),
            pl.BlockSpec((1, 6, Cmax), lambda b: (0, 0, 0)),
        ],
        out_specs=pl.BlockSpec((1, C3, n), lambda b: (b, 0, 0)),
        out_shape=jax.ShapeDtypeStruct((B, C3, n), jnp.float32),
        interpret=interpret,
    )(X, W1p, W2, W3, sb[None])


def _pad_sub(x, mult=8):
    c = x.shape[1]
    cp = ((c + mult - 1) // mult) * mult
    if cp == c:
        return x
    return jnp.pad(x, ((0, 0), (0, cp - c), (0, 0)))


def _gather_points(x, idx):
    return jax.vmap(lambda xi, ii: xi[ii])(x, idx)


def _sa_module(xyz, feat_rows, npoint, radius, nsample, layers, use_small_bq):
    B, N, _ = xyz.shape
    inds = _fps(xyz, npoint)
    new_xyz = _gather_points(xyz, inds)
    del use_small_bq
    idx = _ball_query_xla(radius, nsample, xyz, new_xyz)
    grouped_xyz = (_gather_points(xyz, idx) - new_xyz[:, :, None, :]) / radius
    g = jnp.transpose(grouped_xyz, (0, 3, 1, 2))
    if feat_rows is not None:
        gf = _gather_points(feat_rows, idx)
        g = jnp.concatenate([g, jnp.transpose(gf, (0, 3, 1, 2))], axis=1)
    X = jnp.transpose(g, (0, 1, 3, 2)).reshape(B, g.shape[1], nsample * npoint)
    X = _pad_sub(X)
    f_cols = _mlp_max(X, layers, npoint, nsample)
    return new_xyz, f_cols, inds


def _fp_body(idx_ref, w_ref, f1_ref, f2_ref, w1a_ref, w1b_ref, w2_ref,
             sb_ref, out_ref):
    n1 = idx_ref.shape[1]
    n2 = f2_ref.shape[1]
    flat = jax.lax.broadcasted_iota(jnp.int32, (n1, n2), 1)
    Wh = jnp.zeros((n1, n2), jnp.float32)
    for k in range(3):
        sel = flat == idx_ref[0, :, k:k + 1]
        Wh = Wh + sel.astype(jnp.float32) * w_ref[0, :, k:k + 1]
    interp = jnp.dot(Wh, f2_ref[0], preferred_element_type=jnp.float32,
                    precision=jax.lax.Precision.HIGHEST)
    C1 = w1a_ref.shape[0]
    CO = w2_ref.shape[0]
    s1 = sb_ref[0, 0:1, :C1].reshape(1, C1)
    b1 = sb_ref[0, 1:2, :C1].reshape(1, C1)
    s2 = sb_ref[0, 2:3, :CO].reshape(1, CO)
    b2v = sb_ref[0, 3:4, :CO].reshape(1, CO)
    z = (jnp.dot(f1_ref[0], w1a_ref[...].T, preferred_element_type=jnp.float32,
                    precision=jax.lax.Precision.HIGHEST)
         + jnp.dot(interp, w1b_ref[...].T, preferred_element_type=jnp.float32,
                    precision=jax.lax.Precision.HIGHEST))
    z = jnp.maximum(z / _BN_C * s1 + b1, 0.0)
    z = jnp.dot(z, w2_ref[...].T, preferred_element_type=jnp.float32,
                    precision=jax.lax.Precision.HIGHEST)
    z = jnp.maximum(z / _BN_C * s2 + b2v, 0.0)
    out_ref[0] = z


def _fp_module(xyz1, xyz2, f1_rows, f2_rows, layers, interpret=False):
    B, n1, _ = xyz1.shape
    n2 = xyz2.shape[1]
    d2 = (jnp.sum(xyz1 * xyz1, -1)[:, :, None]
          + jnp.sum(xyz2 * xyz2, -1)[:, None, :]
          - 2.0 * jnp.einsum('bnd,bmd->bnm', xyz1, xyz2))
    negd, idx = jax.lax.top_k(-d2, 3)
    dist = jnp.maximum(-negd, 0.0)
    dist_recip = 1.0 / (dist + 1e-8)
    weight = dist_recip / jnp.sum(dist_recip, -1, keepdims=True)
    (W1, g1, be1), (W2, g2, be2) = layers
    C1 = f1_rows.shape[2]
    C2 = f2_rows.shape[2]
    CO = W2.shape[0]
    CM = W1.shape[0]
    W1a, W1b = W1[:, :C1], W1[:, C1:]
    Cmax = max(CM, CO)
    sb = jnp.zeros((4, Cmax), jnp.float32)
    sb = sb.at[0, :CM].set(g1).at[1, :CM].set(be1)
    sb = sb.at[2, :CO].set(g2).at[3, :CO].set(be2)
    return pl.pallas_call(
        _fp_body,
        grid=(B,),
        in_specs=[
            pl.BlockSpec((1, n1, 3), lambda b: (b, 0, 0)),
            pl.BlockSpec((1, n1, 3), lambda b: (b, 0, 0)),
            pl.BlockSpec((1, n1, C1), lambda b: (b, 0, 0)),
            pl.BlockSpec((1, n2, C2), lambda b: (b, 0, 0)),
            pl.BlockSpec(W1a.shape, lambda b: (0, 0)),
            pl.BlockSpec(W1b.shape, lambda b: (0, 0)),
            pl.BlockSpec(W2.shape, lambda b: (0, 0)),
            pl.BlockSpec((1, 4, Cmax), lambda b: (0, 0, 0)),
        ],
        out_specs=pl.BlockSpec((1, n1, CO), lambda b: (b, 0, 0)),
        out_shape=jax.ShapeDtypeStruct((B, n1, CO), jnp.float32),
        interpret=interpret,
    )(idx, weight, f1_rows, f2_rows, W1a, W1b, W2, sb[None])


def kernel(pointcloud, params):
    xyz = pointcloud[..., 0:3]
    sa1_xyz, sa1_f, sa1_inds = _sa_module(
        xyz, None, 512, 0.04, 64, params['sa1'], use_small_bq=False)
    sa1_rows = jnp.transpose(sa1_f, (0, 2, 1))
    sa2_xyz, sa2_f, _ = _sa_module(
        sa1_xyz, sa1_rows, 256, 0.1, 32, params['sa2'], use_small_bq=True)
    sa2_rows = jnp.transpose(sa2_f, (0, 2, 1))
    sa3_xyz, sa3_f, _ = _sa_module(
        sa2_xyz, sa2_rows, 128, 0.2, 16, params['sa3'], use_small_bq=True)
    sa3_rows = jnp.transpose(sa3_f, (0, 2, 1))
    sa4_xyz, sa4_f, _ = _sa_module(
        sa3_xyz, sa3_rows, 64, 0.3, 16, params['sa4'], use_small_bq=True)
    sa4_rows = jnp.transpose(sa4_f, (0, 2, 1))
    f = _fp_module(sa3_xyz, sa4_xyz, sa3_rows, sa4_rows, params['fp1'])
    f = _fp_module(sa2_xyz, sa3_xyz, sa2_rows, f, params['fp2'])
    f = _fp_module(sa1_xyz, sa2_xyz, sa1_rows, f, params['fp3'])
    return jnp.transpose(f, (0, 2, 1)), sa1_xyz, sa1_inds
